# baseline (speedup 1.0000x reference)
import functools

import jax
import jax.numpy as jnp
from jax import lax
from jax.experimental import pallas as pl
from jax.experimental.pallas import tpu as pltpu

_K = 256
_G = 64


def _item_kernel(it_ref, ij_ref, fl_ref, x_ref, wt_ref, g_ref, bias_ref,
                 out_ref, *, G, K, FG):
    i = pl.program_id(0)
    flags = fl_ref[i]
    base = it_ref[i] * G

    @pl.when((flags & 1) != 0)
    def _():
        out_ref[...] = jnp.zeros_like(out_ref)

    @pl.when((flags & 2) != 0)
    def _():
        xk = x_ref[:, 0, 0, :]
        wkT = wt_ref[:, 0, 0, :]
        gk = g_ref[0]
        gcol = lax.broadcasted_iota(jnp.int32, (FG, K), 0) // 4
        mask = (gk - base) == gcol
        wsel = jnp.broadcast_to(wkT[None, :, :], (G, 4, K)).reshape(FG, K)
        ft = jnp.where(mask, wsel, 0.0)
        out_ref[...] += lax.dot_general(
            xk, ft, (((1,), (1,)), ((), ())),
            preferred_element_type=jnp.float32)

    @pl.when((flags & 4) != 0)
    def _():
        out_ref[...] = jnp.tanh(out_ref[...] + bias_ref[0, 0])


def kernel(x, snp_gene, W, bias):
    B, NS = x.shape
    NG, L = bias.shape
    K, G = _K, _G
    FG = L * G

    NSUB = (NS + K - 1) // K
    NT = (NG + G - 1) // G
    NGP = NT * G
    NI = NSUB + NT

    sg = snp_gene.astype(jnp.int32)
    pad = NSUB * K - NS
    if pad:
        x = jnp.pad(x, ((0, 0), (0, pad)))
        sg = jnp.pad(sg, (0, pad), constant_values=NGP)
        W = jnp.pad(W, ((0, pad), (0, 0)))

    x3 = x.reshape(B, NSUB, 1, K)
    wt3 = W.T.reshape(L, NSUB, 1, K)
    g3 = sg.reshape(NSUB, 1, K)
    biasp = jnp.pad(bias, ((0, NGP - NG), (0, 0))).reshape(NT, 1, FG)

    bnd = (jnp.arange(NT + 1, dtype=jnp.int32) * G).astype(sg.dtype)
    starts = jnp.searchsorted(sg, bnd).astype(jnp.int32)
    jlo = starts[:-1] // K
    jhi = (starts[1:] + K - 1) // K

    cnt = jnp.maximum(jhi - jlo, 1)
    offs = jnp.concatenate([jnp.zeros((1,), jnp.int32),
                            jnp.cumsum(cnt, dtype=jnp.int32)[:-1]])
    pos = jnp.arange(NI, dtype=jnp.int32)
    item_t = jnp.repeat(jnp.arange(NT, dtype=jnp.int32), cnt,
                        total_repeat_length=NI)
    item_j = jnp.clip(jlo[item_t] + pos - offs[item_t], 0, NSUB - 1)
    first = pos == offs[item_t]
    valid = (jlo[item_t] + pos - offs[item_t]) < jhi[item_t]
    last = pos == offs[item_t] + cnt[item_t] - 1
    flags = (first.astype(jnp.int32) + 2 * valid.astype(jnp.int32)
             + 4 * last.astype(jnp.int32))

    grid_spec = pltpu.PrefetchScalarGridSpec(
        num_scalar_prefetch=3,
        grid=(NI,),
        in_specs=[
            pl.BlockSpec((B, 1, 1, K), lambda i, it, ij, fl: (0, ij[i], 0, 0)),
            pl.BlockSpec((L, 1, 1, K), lambda i, it, ij, fl: (0, ij[i], 0, 0)),
            pl.BlockSpec((1, 1, K), lambda i, it, ij, fl: (ij[i], 0, 0)),
            pl.BlockSpec((1, 1, FG), lambda i, it, ij, fl: (it[i], 0, 0)),
        ],
        out_specs=pl.BlockSpec((B, FG), lambda i, it, ij, fl: (0, it[i])),
    )
    out = pl.pallas_call(
        functools.partial(_item_kernel, G=G, K=K, FG=FG),
        grid_spec=grid_spec,
        out_shape=jax.ShapeDtypeStruct((B, NT * FG), jnp.float32),
    )(item_t, item_j, flags, x3, wt3, g3, biasp)
    return out[:, : NG * L]


# speedup vs baseline: 2.3397x; 2.3397x over previous
import functools

import jax
import jax.numpy as jnp
from jax import lax
from jax.experimental import pallas as pl
from jax.experimental.pallas import tpu as pltpu

_K = 256
_G = 64


def _tile_kernel(jlo_ref, jhi_ref, x_ref, wt_ref, g_ref, bias_ref, out_ref,
                 *, G, K, FG):
    t = pl.program_id(0)
    base = t * G
    out_ref[...] = jnp.zeros_like(out_ref)

    gcol = lax.broadcasted_iota(jnp.int32, (FG, K), 0) // 4

    def body(j, carry):
        o = j * K
        xk = x_ref[:, pl.ds(o, K)]
        wkT = wt_ref[:, pl.ds(o, K)]
        gk = g_ref[:, pl.ds(o, K)]
        mask = (gk - base) == gcol
        wsel = jnp.broadcast_to(wkT[None, :, :], (G, 4, K)).reshape(FG, K)
        ft = jnp.where(mask, wsel, 0.0)
        out_ref[...] += lax.dot_general(
            xk, ft, (((1,), (1,)), ((), ())),
            preferred_element_type=jnp.float32)
        return carry

    lax.fori_loop(jlo_ref[t], jhi_ref[t], body, 0)
    out_ref[...] = jnp.tanh(out_ref[...] + bias_ref[0])


def kernel(x, snp_gene, W, bias):
    B, NS = x.shape
    NG, L = bias.shape
    K, G = _K, _G
    FG = L * G

    NSUB = (NS + K - 1) // K
    NT = (NG + G - 1) // G
    NGP = NT * G
    NSP = NSUB * K

    sg = snp_gene.astype(jnp.int32)
    pad = NSP - NS
    if pad:
        x = jnp.pad(x, ((0, 0), (0, pad)))
        sg = jnp.pad(sg, (0, pad), constant_values=NGP)
        W = jnp.pad(W, ((0, pad), (0, 0)))

    wt = W.T
    g2 = sg.reshape(1, NSP)
    biasp = jnp.pad(bias, ((0, NGP - NG), (0, 0))).reshape(NT, 1, FG)

    bnd = (jnp.arange(NT + 1, dtype=jnp.int32) * G).astype(sg.dtype)
    starts = jnp.searchsorted(sg, bnd).astype(jnp.int32)
    jlo = starts[:-1] // K
    jhi = (starts[1:] + K - 1) // K

    grid_spec = pltpu.PrefetchScalarGridSpec(
        num_scalar_prefetch=2,
        grid=(NT,),
        in_specs=[
            pl.BlockSpec((B, NSP), lambda t, lo, hi: (0, 0)),
            pl.BlockSpec((L, NSP), lambda t, lo, hi: (0, 0)),
            pl.BlockSpec((1, NSP), lambda t, lo, hi: (0, 0)),
            pl.BlockSpec((1, 1, FG), lambda t, lo, hi: (t, 0, 0)),
        ],
        out_specs=pl.BlockSpec((B, FG), lambda t, lo, hi: (0, t)),
    )
    out = pl.pallas_call(
        functools.partial(_tile_kernel, G=G, K=K, FG=FG),
        grid_spec=grid_spec,
        out_shape=jax.ShapeDtypeStruct((B, NT * FG), jnp.float32),
    )(jlo, jhi, x, wt, g2, biasp)
    return out[:, : NG * L]


# speedup vs baseline: 2.8846x; 1.2329x over previous
import functools

import jax
import jax.numpy as jnp
from jax import lax
from jax.experimental import pallas as pl
from jax.experimental.pallas import tpu as pltpu

_K = 256
_G = 64


def _tile_kernel(jlo_ref, jhi_ref, x3_ref, wt3_ref, g3_ref, bias_ref, out_ref,
                 *, B, G, K, FG):
    t = pl.program_id(0)
    base = t * G

    gcol = lax.broadcasted_iota(jnp.int32, (FG, K), 0) // 4

    def body(j, acc):
        xk = x3_ref[j]
        wkT = wt3_ref[j]
        gk = g3_ref[pl.ds(j, 1), :]
        mask = (gk - base) == gcol
        wsel = jnp.broadcast_to(wkT[None, :, :], (G, 4, K)).reshape(FG, K)
        ft = jnp.where(mask, wsel, 0.0)
        return acc + lax.dot_general(
            xk, ft, (((1,), (1,)), ((), ())),
            preferred_element_type=jnp.float32)

    acc = lax.fori_loop(jlo_ref[t], jhi_ref[t], body,
                        jnp.zeros((B, FG), jnp.float32))
    out_ref[...] = jnp.tanh(acc + bias_ref[0])


def kernel(x, snp_gene, W, bias):
    B, NS = x.shape
    NG, L = bias.shape
    K, G = _K, _G
    FG = L * G

    NSUB = (NS + K - 1) // K
    NT = (NG + G - 1) // G
    NGP = NT * G

    sg = snp_gene.astype(jnp.int32)
    pad = NSUB * K - NS
    if pad:
        x = jnp.pad(x, ((0, 0), (0, pad)))
        sg = jnp.pad(sg, (0, pad), constant_values=NGP)
        W = jnp.pad(W, ((0, pad), (0, 0)))

    x3 = x.reshape(B, NSUB, K).transpose(1, 0, 2)
    wt3 = W.T.reshape(L, NSUB, K).transpose(1, 0, 2)
    g3 = sg.reshape(NSUB, K)
    biasp = jnp.pad(bias, ((0, NGP - NG), (0, 0))).reshape(NT, 1, FG)

    bnd = (jnp.arange(NT + 1, dtype=jnp.int32) * G).astype(sg.dtype)
    starts = jnp.searchsorted(sg, bnd).astype(jnp.int32)
    jlo = starts[:-1] // K
    jhi = (starts[1:] + K - 1) // K

    grid_spec = pltpu.PrefetchScalarGridSpec(
        num_scalar_prefetch=2,
        grid=(NT,),
        in_specs=[
            pl.BlockSpec(x3.shape, lambda t, lo, hi: (0, 0, 0)),
            pl.BlockSpec(wt3.shape, lambda t, lo, hi: (0, 0, 0)),
            pl.BlockSpec(g3.shape, lambda t, lo, hi: (0, 0)),
            pl.BlockSpec((1, 1, FG), lambda t, lo, hi: (t, 0, 0)),
        ],
        out_specs=pl.BlockSpec((B, FG), lambda t, lo, hi: (0, t)),
    )
    return pl.pallas_call(
        functools.partial(_tile_kernel, B=B, G=G, K=K, FG=FG),
        grid_spec=grid_spec,
        out_shape=jax.ShapeDtypeStruct((B, NG * L), jnp.float32),
    )(jlo, jhi, x3, wt3, g3, biasp)
